# Initial kernel scaffold; baseline (speedup 1.0000x reference)
import jax
import jax.numpy as jnp
from jax.experimental import pallas as pl


def kernel(x, table):
    raise NotImplementedError("write your pallas kernel here")


# speedup vs baseline: 1.8452x; 1.8452x over previous
import functools
import jax
import jax.numpy as jnp
from jax import lax
from jax.experimental import pallas as pl
from jax.experimental.pallas import tpu as pltpu, tpu_sc as plsc

VOCAB = 1000000
D_MODEL = 64
NC, NS = 2, 16
NW = NC * NS
CHUNK = 1024


def _make_gather(B: int, D: int):
    assert B % (NW * CHUNK) == 0
    b_per_w = B // NW
    n_chunks = b_per_w // CHUNK
    mesh = plsc.VectorSubcoreMesh(
        core_axis_name="c", subcore_axis_name="s",
        num_cores=NC, num_subcores=NS)

    @functools.partial(
        pl.kernel, mesh=mesh,
        out_type=jax.ShapeDtypeStruct((B, D), jnp.float32),
        compiler_params=pltpu.CompilerParams(use_tc_tiling_on_sc=False),
        scratch_types=[
            pltpu.VMEM((CHUNK,), jnp.int32),
            pltpu.VMEM((CHUNK, D), jnp.float32),
            pltpu.SemaphoreType.DMA,
        ],
    )
    def k(table_hbm, idx_hbm, out_hbm, idx_v, rows_v, sem):
        wid = lax.axis_index("s") * NC + lax.axis_index("c")
        base = wid * b_per_w

        @pl.loop(0, n_chunks)
        def _(g):
            off = base + g * CHUNK
            pltpu.sync_copy(idx_hbm.at[pl.ds(off, CHUNK)], idx_v)
            pltpu.async_copy(table_hbm.at[idx_v], rows_v, sem).wait()
            pltpu.sync_copy(rows_v, out_hbm.at[pl.ds(off, CHUNK)])

    return k


def kernel(x, table):
    B = x.shape[0] * x.shape[1]
    flat_idx = x.reshape(B).astype(jnp.int32)
    out = _make_gather(B, table.shape[1])(table, flat_idx)
    return out.reshape(x.shape[0], x.shape[1], table.shape[1])


# speedup vs baseline: 1.8713x; 1.0142x over previous
import functools
import jax
import jax.numpy as jnp
from jax import lax
from jax.experimental import pallas as pl
from jax.experimental.pallas import tpu as pltpu, tpu_sc as plsc

VOCAB = 1000000
D_MODEL = 64
NC, NS = 2, 16
NW = NC * NS
CHUNK = 800


def _make_gather(B: int, D: int):
    assert B % (NW * CHUNK) == 0
    b_per_w = B // NW
    n_chunks = b_per_w // CHUNK
    assert n_chunks % 2 == 0
    mesh = plsc.VectorSubcoreMesh(
        core_axis_name="c", subcore_axis_name="s",
        num_cores=NC, num_subcores=NS)

    @functools.partial(
        pl.kernel, mesh=mesh,
        out_type=jax.ShapeDtypeStruct((B, D), jnp.float32),
        compiler_params=pltpu.CompilerParams(use_tc_tiling_on_sc=False),
        scratch_types=[
            pltpu.VMEM((b_per_w,), jnp.int32),
            pltpu.VMEM((CHUNK, D), jnp.float32),
            pltpu.VMEM((CHUNK, D), jnp.float32),
            pltpu.SemaphoreType.DMA,
            pltpu.SemaphoreType.DMA,
            pltpu.SemaphoreType.DMA,
            pltpu.SemaphoreType.DMA,
        ],
    )
    def k(table_hbm, idx_hbm, out_hbm, idx_all, rows0, rows1,
          gsem0, gsem1, ssem0, ssem1):
        wid = lax.axis_index("s") * NC + lax.axis_index("c")
        base = wid * b_per_w
        rows = (rows0, rows1)
        gsem = (gsem0, gsem1)
        ssem = (ssem0, ssem1)

        pltpu.sync_copy(idx_hbm.at[pl.ds(base, b_per_w)], idx_all)

        def gather(c, slot):
            return pltpu.async_copy(
                table_hbm.at[idx_all.at[pl.ds(c * CHUNK, CHUNK)]],
                rows[slot], gsem[slot])

        def gather_wait(c, slot):
            pltpu.make_async_copy(
                table_hbm.at[idx_all.at[pl.ds(c * CHUNK, CHUNK)]],
                rows[slot], gsem[slot]).wait()

        def store(c, slot):
            return pltpu.async_copy(
                rows[slot], out_hbm.at[pl.ds(base + c * CHUNK, CHUNK)],
                ssem[slot])

        def store_wait(c, slot):
            pltpu.make_async_copy(
                rows[slot], out_hbm.at[pl.ds(base + c * CHUNK, CHUNK)],
                ssem[slot]).wait()

        gather(0, 0)

        @pl.loop(0, n_chunks, step=2)
        def _(g):
            for b in range(2):
                c = g + b
                q = 1 - b
                gather_wait(c, b)
                store(c, b)

                @pl.when(c + 1 < n_chunks)
                def _():
                    @pl.when(c > 0)
                    def _():
                        store_wait(c - 1, q)
                    gather(c + 1, q)

        store_wait(n_chunks - 2, 0)
        store_wait(n_chunks - 1, 1)

    return k


def kernel(x, table):
    B = x.shape[0] * x.shape[1]
    flat_idx = x.reshape(B).astype(jnp.int32)
    out = _make_gather(B, table.shape[1])(table, flat_idx)
    return out.reshape(x.shape[0], x.shape[1], table.shape[1])


# speedup vs baseline: 1.8768x; 1.0029x over previous
import functools
import jax
import jax.numpy as jnp
from jax import lax
from jax.experimental import pallas as pl
from jax.experimental.pallas import tpu as pltpu, tpu_sc as plsc

VOCAB = 1000000
D_MODEL = 64
NC, NS = 2, 16
NW = NC * NS
CHUNK = 400
NBUF = 4


def _make_gather(B: int, D: int):
    assert B % (NW * CHUNK) == 0
    b_per_w = B // NW
    n_chunks = b_per_w // CHUNK
    assert n_chunks % NBUF == 0 and n_chunks >= 2 * NBUF
    mesh = plsc.VectorSubcoreMesh(
        core_axis_name="c", subcore_axis_name="s",
        num_cores=NC, num_subcores=NS)

    @functools.partial(
        pl.kernel, mesh=mesh,
        out_type=jax.ShapeDtypeStruct((B, D), jnp.float32),
        compiler_params=pltpu.CompilerParams(use_tc_tiling_on_sc=False),
        scratch_types=[
            pltpu.VMEM((b_per_w,), jnp.int32),
            [pltpu.VMEM((CHUNK, D), jnp.float32)] * NBUF,
            [pltpu.SemaphoreType.DMA] * NBUF,
            [pltpu.SemaphoreType.DMA] * NBUF,
        ],
    )
    def k(table_hbm, idx_hbm, out_hbm, idx_all, rows, gsem, ssem):
        wid = lax.axis_index("s") * NC + lax.axis_index("c")
        base = wid * b_per_w

        pltpu.sync_copy(idx_hbm.at[pl.ds(base, b_per_w)], idx_all)

        def gather(c, slot):
            return pltpu.async_copy(
                table_hbm.at[idx_all.at[pl.ds(c * CHUNK, CHUNK)]],
                rows[slot], gsem[slot])

        def gather_wait(c, slot):
            pltpu.make_async_copy(
                table_hbm.at[idx_all.at[pl.ds(c * CHUNK, CHUNK)]],
                rows[slot], gsem[slot]).wait()

        def store(c, slot):
            return pltpu.async_copy(
                rows[slot], out_hbm.at[pl.ds(base + c * CHUNK, CHUNK)],
                ssem[slot])

        def store_wait(c, slot):
            pltpu.make_async_copy(
                rows[slot], out_hbm.at[pl.ds(base + c * CHUNK, CHUNK)],
                ssem[slot]).wait()

        for b in range(NBUF - 1):
            gather(b, b)

        @pl.loop(0, n_chunks, step=NBUF)
        def _(g):
            for b in range(NBUF):
                c = g + b
                bp = (b - 1) % NBUF
                gather_wait(c, b)
                store(c, b)

                @pl.when(c + NBUF - 1 < n_chunks)
                def _():
                    @pl.when(c > 0)
                    def _():
                        store_wait(c - 1, bp)
                    gather(c + NBUF - 1, bp)

        for i in range(NBUF):
            c = n_chunks - NBUF + i
            store_wait(c, c % NBUF)

    return k


def kernel(x, table):
    B = x.shape[0] * x.shape[1]
    flat_idx = x.reshape(B).astype(jnp.int32)
    out = _make_gather(B, table.shape[1])(table, flat_idx)
    return out.reshape(x.shape[0], x.shape[1], table.shape[1])
